# Initial kernel scaffold; baseline (speedup 1.0000x reference)
import jax
import jax.numpy as jnp
from jax.experimental import pallas as pl


def kernel(x, edge_index, W1, b1, g1, be1, m1, v1, W2, b2, Wp, bp):
    raise NotImplementedError("write your pallas kernel here")


# speedup vs baseline: 21.9052x; 21.9052x over previous
import functools

import jax
import jax.numpy as jnp
from jax import lax
from jax.experimental import pallas as pl
from jax.experimental.pallas import tpu as pltpu
from jax.experimental.pallas import tpu_sc as plsc

_NC = 2
_NS = 16
_NW = _NC * _NS
_CHUNK = 128
_EPS = 1e-5


def _sc_mesh():
    return plsc.VectorSubcoreMesh(
        core_axis_name="c", subcore_axis_name="s",
        num_cores=_NC, num_subcores=_NS)


def _deg_hist(didx, ones_h, zeros1, nacc, c_chunks):
    zr = nacc // _NS

    def body(didx_h, ones_hb, zeros_hb, deg_out, idx_v, ones_v, deg_sh):
        c = lax.axis_index("c")
        s = lax.axis_index("s")
        w = c * _NS + s
        pltpu.sync_copy(didx_h.at[w], idx_v)
        pltpu.sync_copy(ones_hb, ones_v)
        pltpu.sync_copy(zeros_hb.at[pl.ds(s * zr, zr)],
                        deg_sh.at[pl.ds(s * zr, zr)])
        plsc.subcore_barrier()

        def step(i, carry):
            pltpu.sync_copy(ones_v, deg_sh.at[idx_v.at[i]], add=True)
            return carry

        lax.fori_loop(0, c_chunks, step, 0)
        plsc.subcore_barrier()
        pltpu.sync_copy(deg_sh.at[pl.ds(s * zr, zr)],
                        deg_out.at[pl.ds(c * nacc + s * zr, zr)])

    return pl.kernel(
        body,
        out_type=jax.ShapeDtypeStruct((_NC * nacc,), jnp.float32),
        mesh=_sc_mesh(),
        scratch_types=[
            pltpu.VMEM((c_chunks, _CHUNK), jnp.int32),
            pltpu.VMEM((_CHUNK,), jnp.float32),
            pltpu.VMEM_SHARED((nacc,), jnp.float32),
        ],
    )(didx, ones_h, zeros1)


def _edge_scatter(hsrc, sidx, didx, zeros2, nacc, c_chunks, d):
    zr = nacc // _NS

    def body(h_hb, sidx_h, didx_h, zeros_hb, out_hb,
             sidx_v, didx_v, rows_v, acc_sh, sem):
        c = lax.axis_index("c")
        s = lax.axis_index("s")
        w = c * _NS + s
        pltpu.sync_copy(sidx_h.at[w], sidx_v)
        pltpu.sync_copy(didx_h.at[w], didx_v)
        pltpu.sync_copy(zeros_hb.at[pl.ds(s * zr, zr)],
                        acc_sh.at[pl.ds(s * zr, zr)])
        plsc.subcore_barrier()

        def step(i, carry):
            pltpu.async_copy(h_hb.at[sidx_v.at[i]], rows_v, sem).wait()
            pltpu.sync_copy(rows_v, acc_sh.at[didx_v.at[i]], add=True)
            return carry

        lax.fori_loop(0, c_chunks, step, 0)
        plsc.subcore_barrier()
        pltpu.sync_copy(acc_sh.at[pl.ds(s * zr, zr)],
                        out_hb.at[pl.ds(c * nacc + s * zr, zr)])

    return pl.kernel(
        body,
        out_type=jax.ShapeDtypeStruct((_NC * nacc, d), jnp.float32),
        mesh=_sc_mesh(),
        scratch_types=[
            pltpu.VMEM((c_chunks, _CHUNK), jnp.int32),
            pltpu.VMEM((c_chunks, _CHUNK), jnp.int32),
            pltpu.VMEM((_CHUNK, d), jnp.float32),
            pltpu.VMEM_SHARED((nacc, d), jnp.float32),
            pltpu.SemaphoreType.DMA,
        ],
    )(hsrc, sidx, didx, zeros2)


def _tc1_body(xp_ref, w1_ref, d0_ref, d1_ref, h1p_ref, dinv_ref):
    dinv = lax.rsqrt(1.0 + d0_ref[...] + d1_ref[...])
    h = jnp.dot(xp_ref[...], w1_ref[...], preferred_element_type=jnp.float32)
    h1p_ref[...] = h * dinv
    dinv_ref[...] = dinv


def _tc1(xp, w1, d0, d1, nacc, d_in, d_hid, br):
    grid = (nacc // br,)
    return pl.pallas_call(
        _tc1_body,
        grid=grid,
        in_specs=[
            pl.BlockSpec((br, d_in), lambda i: (i, 0)),
            pl.BlockSpec((d_in, d_hid), lambda i: (0, 0)),
            pl.BlockSpec((br, 1), lambda i: (i, 0)),
            pl.BlockSpec((br, 1), lambda i: (i, 0)),
        ],
        out_specs=[
            pl.BlockSpec((br, d_hid), lambda i: (i, 0)),
            pl.BlockSpec((br, 1), lambda i: (i, 0)),
        ],
        out_shape=[
            jax.ShapeDtypeStruct((nacc, d_hid), jnp.float32),
            jax.ShapeDtypeStruct((nacc, 1), jnp.float32),
        ],
    )(xp, w1, d0, d1)


def _tc2_body(a0_ref, a1_ref, h1p_ref, dinv_ref, w2_ref,
              b1_ref, g1_ref, be1_ref, m1_ref, v1_ref,
              x1_ref, h2p_ref):
    dinv = dinv_ref[...]
    s = dinv * (a0_ref[...] + a1_ref[...] + h1p_ref[...]) + b1_ref[...]
    scale = g1_ref[...] * lax.rsqrt(v1_ref[...] + _EPS)
    x1 = jnp.maximum((s - m1_ref[...]) * scale + be1_ref[...], 0.0)
    x1_ref[...] = x1
    h2 = jnp.dot(x1, w2_ref[...], preferred_element_type=jnp.float32)
    h2p_ref[...] = h2 * dinv


def _tc2(accp, h1p, dinv, w2, b1, g1, be1, m1, v1, nacc, d_hid, br):
    grid = (nacc // br,)
    vec = pl.BlockSpec((1, d_hid), lambda i: (0, 0))
    return pl.pallas_call(
        _tc2_body,
        grid=grid,
        in_specs=[
            pl.BlockSpec((None, br, d_hid), lambda i: (0, i, 0)),
            pl.BlockSpec((None, br, d_hid), lambda i: (1, i, 0)),
            pl.BlockSpec((br, d_hid), lambda i: (i, 0)),
            pl.BlockSpec((br, 1), lambda i: (i, 0)),
            pl.BlockSpec((d_hid, d_hid), lambda i: (0, 0)),
            vec, vec, vec, vec, vec,
        ],
        out_specs=[
            pl.BlockSpec((br, d_hid), lambda i: (i, 0)),
            pl.BlockSpec((br, d_hid), lambda i: (i, 0)),
        ],
        out_shape=[
            jax.ShapeDtypeStruct((nacc, d_hid), jnp.float32),
            jax.ShapeDtypeStruct((nacc, d_hid), jnp.float32),
        ],
    )(accp, accp, h1p, dinv, w2, b1, g1, be1, m1, v1)


def _tc3_body(a0_ref, a1_ref, h2p_ref, dinv_ref, x1_ref, wp_ref,
              b2_ref, bp_ref, out_ref):
    dinv = dinv_ref[...]
    x2 = dinv * (a0_ref[...] + a1_ref[...] + h2p_ref[...]) + b2_ref[...]
    j = jnp.maximum(x1_ref[...], x2)
    out_ref[...] = (jnp.dot(j, wp_ref[...], preferred_element_type=jnp.float32)
                    + bp_ref[...])


def _tc3(accp, h2p, dinv, x1, wp, b2, bp, n, d_hid, d_out, br):
    grid = (n // br,)
    return pl.pallas_call(
        _tc3_body,
        grid=grid,
        in_specs=[
            pl.BlockSpec((None, br, d_hid), lambda i: (0, i, 0)),
            pl.BlockSpec((None, br, d_hid), lambda i: (1, i, 0)),
            pl.BlockSpec((br, d_hid), lambda i: (i, 0)),
            pl.BlockSpec((br, 1), lambda i: (i, 0)),
            pl.BlockSpec((br, d_hid), lambda i: (i, 0)),
            pl.BlockSpec((d_hid, d_out), lambda i: (0, 0)),
            pl.BlockSpec((1, d_hid), lambda i: (0, 0)),
            pl.BlockSpec((1, d_out), lambda i: (0, 0)),
        ],
        out_specs=pl.BlockSpec((br, d_out), lambda i: (i, 0)),
        out_shape=jax.ShapeDtypeStruct((n, d_out), jnp.float32),
    )(accp, accp, h2p, dinv, x1, wp, b2, bp)


def kernel(x, edge_index, W1, b1, g1, be1, m1, v1, W2, b2, Wp, bp):
    n, d_in = x.shape
    d_hid = W1.shape[1]
    d_out = Wp.shape[1]
    e = edge_index.shape[1]

    zr16 = _NS * 8
    nacc = ((n + 16 + zr16 * _NS - 1) // (zr16 * _NS)) * (zr16 * _NS)
    c_chunks = -(-e // (_NW * _CHUNK))
    e_pad = _NW * c_chunks * _CHUNK

    pad = e_pad - e
    pad_idx = (n + (jnp.arange(pad, dtype=jnp.int32) % 16))
    srcp = jnp.concatenate([edge_index[0], pad_idx]).reshape(_NW, c_chunks, _CHUNK)
    dstp = jnp.concatenate([edge_index[1], pad_idx]).reshape(_NW, c_chunks, _CHUNK)

    zeros1 = jnp.zeros((nacc,), jnp.float32)
    zeros2 = jnp.zeros((nacc, d_hid), jnp.float32)
    ones_h = jnp.ones((_CHUNK,), jnp.float32)

    degp = _deg_hist(dstp, ones_h, zeros1, nacc, c_chunks)
    d0 = degp[:nacc].reshape(nacc, 1)
    d1 = degp[nacc:].reshape(nacc, 1)

    xp = jnp.pad(x, ((0, nacc - n), (0, 0)))
    br = nacc // 10
    h1p, dinv = _tc1(xp, W1, d0, d1, nacc, d_in, d_hid, br)

    acc1 = _edge_scatter(h1p, srcp, dstp, zeros2, nacc, c_chunks, d_hid)
    acc1 = acc1.reshape(_NC, nacc, d_hid)

    x1, h2p = _tc2(acc1, h1p, dinv, W2,
                   b1.reshape(1, d_hid), g1.reshape(1, d_hid),
                   be1.reshape(1, d_hid), m1.reshape(1, d_hid),
                   v1.reshape(1, d_hid), nacc, d_hid, br)

    acc2 = _edge_scatter(h2p, srcp, dstp, zeros2, nacc, c_chunks, d_hid)
    acc2 = acc2.reshape(_NC, nacc, d_hid)

    br3 = n // 10
    out = _tc3(acc2, h2p, dinv, x1, Wp,
               b2.reshape(1, d_hid), bp.reshape(1, d_out), n, d_hid, d_out, br3)
    return out
